# Initial kernel scaffold; baseline (speedup 1.0000x reference)
import jax
import jax.numpy as jnp
from jax.experimental import pallas as pl


def kernel(x, codebook):
    raise NotImplementedError("write your pallas kernel here")


# speedup vs baseline: 1.6066x; 1.6066x over previous
import jax
import jax.numpy as jnp
from jax.experimental import pallas as pl

N_TOKENS = 131072
EMBED_DIM = 32
N_LATENTS = 1024
BLOCK = 1024


def _vq_block_kernel(x_ref, cbt_ref, cb_ref, z_ref, q_ref, loss_ref):
    i = pl.program_id(0)
    xb = x_ref[...]
    cbt = cbt_ref[...]
    cb = cb_ref[...]

    scores = jax.lax.dot_general(
        xb, cbt, (((1,), (0,)), ((), ())),
        preferred_element_type=jnp.float32,
    )
    x_sq = jnp.sum(xb * xb, axis=1, keepdims=True)
    c_sq = jnp.sum(cb * cb, axis=1)
    dist = x_sq + c_sq[None, :] - 2.0 * scores

    dmin = jnp.min(dist, axis=1, keepdims=True)
    kidx = jax.lax.broadcasted_iota(jnp.int32, (BLOCK, N_LATENTS), 1)
    z = jnp.min(jnp.where(dist == dmin, kidx, N_LATENTS), axis=1)
    z = z.astype(jnp.int32)

    onehot = (kidx == z[:, None]).astype(jnp.float32)
    q = jax.lax.dot_general(
        onehot, cb, (((1,), (0,)), ((), ())),
        preferred_element_type=jnp.float32,
    )

    z_ref[...] = z
    q_ref[...] = xb + (q - xb)

    part = jnp.sum((q - xb) ** 2)[None, None]

    @pl.when(i == 0)
    def _():
        loss_ref[...] = jnp.zeros_like(loss_ref)

    loss_ref[...] += part


@jax.jit
def kernel(x, codebook):
    n, d = x.shape
    k = codebook.shape[0]
    grid = n // BLOCK
    cbt = codebook.T

    z, q, loss_sum = pl.pallas_call(
        _vq_block_kernel,
        grid=(grid,),
        in_specs=[
            pl.BlockSpec((BLOCK, d), lambda i: (i, 0)),
            pl.BlockSpec((d, k), lambda i: (0, 0)),
            pl.BlockSpec((k, d), lambda i: (0, 0)),
        ],
        out_specs=[
            pl.BlockSpec((BLOCK,), lambda i: (i,)),
            pl.BlockSpec((BLOCK, d), lambda i: (i, 0)),
            pl.BlockSpec((1, 1), lambda i: (0, 0)),
        ],
        out_shape=[
            jax.ShapeDtypeStruct((n,), jnp.int32),
            jax.ShapeDtypeStruct((n, d), jnp.float32),
            jax.ShapeDtypeStruct((1, 1), jnp.float32),
        ],
    )(x, cbt, codebook)

    loss = loss_sum[0, 0] / (n * d)
    return (z, q, (loss, loss))
